# baseline (speedup 1.0000x reference)
import jax
import jax.numpy as jnp
from jax.experimental import pallas as pl
from jax.experimental.pallas import tpu as pltpu

_ROWS = 2048
_VOCAB = 100000
_Q = 4
_R = 8
_BAND = _ROWS // _Q
_STEPS = _BAND // _R
_W = 512
_NCHUNK = _VOCAB // _W
_TAIL = _VOCAB - _NCHUNK * _W
_BIG = 2**30


def _band_reduce(x_ref):
    m = x_ref[0, :, 0:_W]
    bi = jnp.zeros((_R, _W), jnp.int32)
    for k in range(1, _NCHUNK):
        xk = x_ref[0, :, _W * k:_W * (k + 1)]
        gt = xk > m
        m = jnp.maximum(m, xk)
        bi = jnp.where(gt, jnp.int32(k), bi)
    xt = x_ref[0, :, _NCHUNK * _W:_VOCAB]

    maxx = jnp.maximum(jnp.max(m, axis=-1), jnp.max(xt, axis=-1))
    lane = jax.lax.broadcasted_iota(jnp.int32, (_R, _W), 1)
    cand = jnp.where(m == maxx[:, None], bi * _W + lane, _BIG)
    lane_t = jax.lax.broadcasted_iota(jnp.int32, (_R, _TAIL), 1)
    cand_t = jnp.where(xt == maxx[:, None], _NCHUNK * _W + lane_t, _BIG)
    ids = jnp.minimum(jnp.min(cand, axis=-1), jnp.min(cand_t, axis=-1))
    return maxx, ids


def _fwd_kernel(loss_ref, *refs):
    x_refs = refs[:_Q]
    loss_out_ref = refs[_Q]
    max_refs = refs[_Q + 1:2 * _Q + 1]
    ids_refs = refs[2 * _Q + 1:]
    for q in range(_Q):
        maxx, ids = _band_reduce(x_refs[q])
        max_refs[q][0, :, 0] = maxx
        ids_refs[q][0, :, 0] = ids
    loss_out_ref[0, 0] = loss_ref[0, 0]


def kernel(loss, logits):
    loss2d = loss.reshape(1, 1)

    def _in_map(j):
        return lambda i: (0, i + j * _STEPS, 0)

    def _out_map(i):
        return (0, i, 0)

    outs = pl.pallas_call(
        _fwd_kernel,
        grid=(_STEPS,),
        in_specs=[pl.BlockSpec(memory_space=pltpu.SMEM)]
        + [pl.BlockSpec((1, _R, _VOCAB), _in_map(j)) for j in range(_Q)],
        out_specs=[pl.BlockSpec(memory_space=pltpu.SMEM)]
        + [pl.BlockSpec((1, _R, 1), _out_map) for _ in range(2 * _Q)],
        out_shape=[jax.ShapeDtypeStruct((1, 1), jnp.float32)]
        + [jax.ShapeDtypeStruct((1, _BAND, 1), jnp.float32) for _ in range(_Q)]
        + [jax.ShapeDtypeStruct((1, _BAND, 1), jnp.int32) for _ in range(_Q)],
    )(loss2d, *([logits] * _Q))
    return outs[0].reshape(())
